# initial kernel scaffold (unmeasured)
import functools

import jax
import jax.numpy as jnp
from jax import lax
from jax.experimental import pallas as pl
from jax.experimental.pallas import tpu as pltpu

N_DEV = 8
M_PER = 128
K = 1024
N_PER = 128

_GELU_C = 0.7978845608028654


def _gelu(y):
    return 0.5 * y * (1.0 + jnp.tanh(_GELU_C * (y + 0.044715 * y * y * y)))


def _ring(s):
    s = s % N_DEV
    return jnp.where(s < 4, s, 11 - s)


def kernel(x, w_mat):
    def body(x_ref, w_ref, out_ref, comm_ref, send_sems, recv_sems):
        my_pos = lax.axis_index("i")
        my_slot = _ring(my_pos)
        right = _ring(_ring(my_slot + 1))
        left = _ring(_ring(my_slot - 1))

        barrier_sem = pltpu.get_barrier_semaphore()
        for nbr in (left, right):
            pl.semaphore_signal(
                barrier_sem, inc=1,
                device_id=(nbr,), device_id_type=pl.DeviceIdType.MESH,
            )
        pl.semaphore_wait(barrier_sem, 2)

        for h in range(N_DEV - 1):
            o_send = _ring(my_slot - h)
            o_recv = _ring(my_slot - h - 1)
            src = x_ref if h == 0 else comm_ref.at[o_send]
            send = pltpu.make_async_remote_copy(
                src_ref=src,
                dst_ref=comm_ref.at[o_send],
                send_sem=send_sems.at[h],
                recv_sem=recv_sems.at[h],
                device_id=(right,),
                device_id_type=pl.DeviceIdType.MESH,
            )
            send.start()

            if h == 0:
                y = jnp.dot(x_ref[...], w_ref[...],
                            preferred_element_type=jnp.float32)
                out_ref[pl.ds(my_pos * M_PER, M_PER), :] = _gelu(y)

            recv = pltpu.make_async_remote_copy(
                src_ref=comm_ref.at[o_recv],
                dst_ref=comm_ref.at[o_recv],
                send_sem=send_sems.at[h],
                recv_sem=recv_sems.at[h],
                device_id=(left,),
                device_id_type=pl.DeviceIdType.MESH,
            )
            recv.wait_recv()

            y = jnp.dot(comm_ref[o_recv], w_ref[...],
                        preferred_element_type=jnp.float32)
            out_ref[pl.ds(o_recv * M_PER, M_PER), :] = _gelu(y)

            send.wait_send()

    return pl.pallas_call(
        body,
        out_shape=jax.ShapeDtypeStruct((N_DEV * M_PER, N_PER), jnp.float32),
        in_specs=[
            pl.BlockSpec(memory_space=pltpu.VMEM),
            pl.BlockSpec(memory_space=pltpu.VMEM),
        ],
        out_specs=pl.BlockSpec(memory_space=pltpu.VMEM),
        scratch_shapes=[
            pltpu.VMEM((N_DEV, M_PER, K), jnp.float32),
            pltpu.SemaphoreType.DMA((N_DEV - 1,)),
            pltpu.SemaphoreType.DMA((N_DEV - 1,)),
        ],
        compiler_params=pltpu.CompilerParams(collective_id=0),
    )(x, w_mat)


# baseline (device time: 59229 ns/iter reference)
import functools

import jax
import jax.numpy as jnp
from jax import lax
from jax.experimental import pallas as pl
from jax.experimental.pallas import tpu as pltpu

N_DEV = 8
M_PER = 128
K = 1024
N_PER = 128

_GELU_C = 0.7978845608028654


def _gelu(y):
    return 0.5 * y * (1.0 + jnp.tanh(_GELU_C * (y + 0.044715 * y * y * y)))


def _ring(s):
    s = s % N_DEV
    return jnp.where(s < 4, s, 11 - s)


def kernel(x, w_mat):
    def body(x_ref, w_ref, out_ref, comm_ref, send_sems, recv_sems):
        my_pos = lax.axis_index("i")
        my_slot = _ring(my_pos)
        right = _ring(my_slot + 1)
        left = _ring(my_slot - 1)

        barrier_sem = pltpu.get_barrier_semaphore()
        for nbr in (left, right):
            pl.semaphore_signal(
                barrier_sem, inc=1,
                device_id=(nbr,), device_id_type=pl.DeviceIdType.MESH,
            )
        pl.semaphore_wait(barrier_sem, 2)

        for h in range(N_DEV - 1):
            o_send = _ring(my_slot - h)
            o_recv = _ring(my_slot - h - 1)
            src = x_ref if h == 0 else comm_ref.at[o_send]
            send = pltpu.make_async_remote_copy(
                src_ref=src,
                dst_ref=comm_ref.at[o_send],
                send_sem=send_sems.at[h],
                recv_sem=recv_sems.at[h],
                device_id=(right,),
                device_id_type=pl.DeviceIdType.MESH,
            )
            send.start()

            if h == 0:
                y = jnp.dot(x_ref[...], w_ref[...],
                            preferred_element_type=jnp.float32)
                out_ref[pl.ds(my_pos * M_PER, M_PER), :] = _gelu(y)

            recv = pltpu.make_async_remote_copy(
                src_ref=comm_ref.at[o_recv],
                dst_ref=comm_ref.at[o_recv],
                send_sem=send_sems.at[h],
                recv_sem=recv_sems.at[h],
                device_id=(left,),
                device_id_type=pl.DeviceIdType.MESH,
            )
            recv.wait_recv()

            y = jnp.dot(comm_ref[o_recv], w_ref[...],
                        preferred_element_type=jnp.float32)
            out_ref[pl.ds(o_recv * M_PER, M_PER), :] = _gelu(y)

            send.wait_send()

    return pl.pallas_call(
        body,
        out_shape=jax.ShapeDtypeStruct((N_DEV * M_PER, N_PER), jnp.float32),
        in_specs=[
            pl.BlockSpec(memory_space=pltpu.VMEM),
            pl.BlockSpec(memory_space=pltpu.VMEM),
        ],
        out_specs=pl.BlockSpec(memory_space=pltpu.VMEM),
        scratch_shapes=[
            pltpu.VMEM((N_DEV, M_PER, K), jnp.float32),
            pltpu.SemaphoreType.DMA((N_DEV - 1,)),
            pltpu.SemaphoreType.DMA((N_DEV - 1,)),
        ],
        compiler_params=pltpu.CompilerParams(collective_id=0),
    )(x, w_mat)


# device time: 38635 ns/iter; 1.5330x vs baseline; 1.5330x over previous
import jax
import jax.numpy as jnp
from jax import lax
from jax.experimental import pallas as pl
from jax.experimental.pallas import tpu as pltpu

N_DEV = 8
M_PER = 128
K = 1024
N_PER = 128
HOPS = N_DEV - 1

STREAMS_PER_DIR = 1
N_STREAMS = 2 * STREAMS_PER_DIR
ROWS_PER_STREAM = M_PER // N_STREAMS

_GELU_C = 0.7978845608028654


def _gelu(y):
    return 0.5 * y * (1.0 + jnp.tanh(_GELU_C * (y + 0.044715 * y * y * y)))


def _ring(s):
    s = s % N_DEV
    return jnp.where(s < 4, s, 11 - s)


def kernel(x, w_mat):
    def body(x_ref, w_ref, out_ref, *scr):
        comm = scr[:N_STREAMS]
        send_sems = scr[N_STREAMS:2 * N_STREAMS]
        recv_sems = scr[2 * N_STREAMS:3 * N_STREAMS]

        my_pos = lax.axis_index("i")
        my_slot = _ring(my_pos)
        right = _ring(my_slot + 1)
        left = _ring(my_slot - 1)

        for st in range(N_STREAMS):
            off = st * ROWS_PER_STREAM
            comm[st][my_pos, :, :] = x_ref[off:off + ROWS_PER_STREAM, :]

        barrier_sem = pltpu.get_barrier_semaphore()
        for nbr in (left, right):
            pl.semaphore_signal(
                barrier_sem, inc=1,
                device_id=(nbr,), device_id_type=pl.DeviceIdType.MESH,
            )
        pl.semaphore_wait(barrier_sem, 2)

        pending_sends = []

        def start_send(st, hop, origin):
            cw = st < STREAMS_PER_DIR
            desc = pltpu.make_async_remote_copy(
                src_ref=comm[st].at[origin],
                dst_ref=comm[st].at[origin],
                send_sem=send_sems[st].at[hop],
                recv_sem=recv_sems[st].at[hop],
                device_id=(right if cw else left,),
                device_id_type=pl.DeviceIdType.MESH,
            )
            desc.start()
            pending_sends.append(desc)

        for st in range(N_STREAMS):
            start_send(st, 0, my_pos)

        y = jnp.dot(x_ref[...], w_ref[...], preferred_element_type=jnp.float32)
        out_ref[pl.ds(my_pos * M_PER, M_PER), :] = _gelu(y)

        for h in range(HOPS):
            for st in range(N_STREAMS):
                cw = st < STREAMS_PER_DIR
                o_recv = _ring(my_slot - h - 1) if cw else _ring(my_slot + h + 1)
                recv = pltpu.make_async_remote_copy(
                    src_ref=comm[st].at[o_recv],
                    dst_ref=comm[st].at[o_recv],
                    send_sem=send_sems[st].at[h],
                    recv_sem=recv_sems[st].at[h],
                    device_id=(left if cw else right,),
                    device_id_type=pl.DeviceIdType.MESH,
                )
                recv.wait_recv()
                if h < HOPS - 1:
                    start_send(st, h + 1, o_recv)
                y = jnp.dot(comm[st][o_recv], w_ref[...],
                            preferred_element_type=jnp.float32)
                row = o_recv * M_PER + st * ROWS_PER_STREAM
                out_ref[pl.ds(row, ROWS_PER_STREAM), :] = _gelu(y)

        for desc in pending_sends:
            desc.wait_send()

    scratch = (
        [pltpu.VMEM((N_DEV, ROWS_PER_STREAM, K), jnp.float32)] * N_STREAMS
        + [pltpu.SemaphoreType.DMA((HOPS,))] * N_STREAMS
        + [pltpu.SemaphoreType.DMA((HOPS,))] * N_STREAMS
    )
    return pl.pallas_call(
        body,
        out_shape=jax.ShapeDtypeStruct((N_DEV * M_PER, N_PER), jnp.float32),
        in_specs=[
            pl.BlockSpec(memory_space=pltpu.VMEM),
            pl.BlockSpec(memory_space=pltpu.VMEM),
        ],
        out_specs=pl.BlockSpec(memory_space=pltpu.VMEM),
        scratch_shapes=scratch,
        compiler_params=pltpu.CompilerParams(collective_id=0),
    )(x, w_mat)


# device time: 32298 ns/iter; 1.8338x vs baseline; 1.1962x over previous
import jax
import jax.numpy as jnp
from jax import lax
from jax.experimental import pallas as pl
from jax.experimental.pallas import tpu as pltpu

N_DEV = 8
M_PER = 128
K = 1024
N_PER = 128
HOPS = N_DEV - 1

STREAMS_PER_DIR = 2
N_STREAMS = 2 * STREAMS_PER_DIR
ROWS_PER_STREAM = M_PER // N_STREAMS

_GELU_C = 0.7978845608028654


def _gelu(y):
    return 0.5 * y * (1.0 + jnp.tanh(_GELU_C * (y + 0.044715 * y * y * y)))


def _ring(s):
    s = s % N_DEV
    return jnp.where(s < 4, s, 11 - s)


def kernel(x, w_mat):
    def body(x_ref, w_ref, out_ref, *scr):
        comm = scr[:N_STREAMS]
        send_sems = scr[N_STREAMS:2 * N_STREAMS]
        recv_sems = scr[2 * N_STREAMS:3 * N_STREAMS]

        my_pos = lax.axis_index("i")
        my_slot = _ring(my_pos)
        right = _ring(my_slot + 1)
        left = _ring(my_slot - 1)

        for st in range(N_STREAMS):
            off = st * ROWS_PER_STREAM
            comm[st][my_pos, :, :] = x_ref[off:off + ROWS_PER_STREAM, :]

        barrier_sem = pltpu.get_barrier_semaphore()
        for nbr in (left, right):
            pl.semaphore_signal(
                barrier_sem, inc=1,
                device_id=(nbr,), device_id_type=pl.DeviceIdType.MESH,
            )
        pl.semaphore_wait(barrier_sem, 2)

        pending_sends = []

        def start_send(st, hop, origin):
            cw = st < STREAMS_PER_DIR
            desc = pltpu.make_async_remote_copy(
                src_ref=comm[st].at[origin],
                dst_ref=comm[st].at[origin],
                send_sem=send_sems[st].at[hop],
                recv_sem=recv_sems[st].at[hop],
                device_id=(right if cw else left,),
                device_id_type=pl.DeviceIdType.MESH,
            )
            desc.start()
            pending_sends.append(desc)

        for st in range(N_STREAMS):
            start_send(st, 0, my_pos)

        y = jnp.dot(x_ref[...], w_ref[...], preferred_element_type=jnp.float32)
        out_ref[pl.ds(my_pos * M_PER, M_PER), :] = _gelu(y)

        for h in range(HOPS):
            for st in range(N_STREAMS):
                cw = st < STREAMS_PER_DIR
                o_recv = _ring(my_slot - h - 1) if cw else _ring(my_slot + h + 1)
                recv = pltpu.make_async_remote_copy(
                    src_ref=comm[st].at[o_recv],
                    dst_ref=comm[st].at[o_recv],
                    send_sem=send_sems[st].at[h],
                    recv_sem=recv_sems[st].at[h],
                    device_id=(left if cw else right,),
                    device_id_type=pl.DeviceIdType.MESH,
                )
                recv.wait_recv()
                if h < HOPS - 1:
                    start_send(st, h + 1, o_recv)
                y = jnp.dot(comm[st][o_recv], w_ref[...],
                            preferred_element_type=jnp.float32)
                row = o_recv * M_PER + st * ROWS_PER_STREAM
                out_ref[pl.ds(row, ROWS_PER_STREAM), :] = _gelu(y)

        for desc in pending_sends:
            desc.wait_send()

    scratch = (
        [pltpu.VMEM((N_DEV, ROWS_PER_STREAM, K), jnp.float32)] * N_STREAMS
        + [pltpu.SemaphoreType.DMA((HOPS,))] * N_STREAMS
        + [pltpu.SemaphoreType.DMA((HOPS,))] * N_STREAMS
    )
    return pl.pallas_call(
        body,
        out_shape=jax.ShapeDtypeStruct((N_DEV * M_PER, N_PER), jnp.float32),
        in_specs=[
            pl.BlockSpec(memory_space=pltpu.VMEM),
            pl.BlockSpec(memory_space=pltpu.VMEM),
        ],
        out_specs=pl.BlockSpec(memory_space=pltpu.VMEM),
        scratch_shapes=scratch,
        compiler_params=pltpu.CompilerParams(collective_id=0),
    )(x, w_mat)


# device time: 31523 ns/iter; 1.8789x vs baseline; 1.0246x over previous
import jax
import jax.numpy as jnp
from jax import lax
from jax.experimental import pallas as pl
from jax.experimental.pallas import tpu as pltpu

N_DEV = 8
M_PER = 128
K = 1024
N_PER = 128
HOPS = N_DEV - 1

STREAMS_PER_DIR = 4
N_STREAMS = 2 * STREAMS_PER_DIR
ROWS_PER_STREAM = M_PER // N_STREAMS

_GELU_C = 0.7978845608028654


def _gelu(y):
    return 0.5 * y * (1.0 + jnp.tanh(_GELU_C * (y + 0.044715 * y * y * y)))


def _ring(s):
    s = s % N_DEV
    return jnp.where(s < 4, s, 11 - s)


def kernel(x, w_mat):
    def body(x_ref, w_ref, out_ref, *scr):
        comm = scr[:N_STREAMS]
        send_sems = scr[N_STREAMS:2 * N_STREAMS]
        recv_sems = scr[2 * N_STREAMS:3 * N_STREAMS]

        my_pos = lax.axis_index("i")
        my_slot = _ring(my_pos)
        right = _ring(my_slot + 1)
        left = _ring(my_slot - 1)

        for st in range(N_STREAMS):
            off = st * ROWS_PER_STREAM
            comm[st][my_pos, :, :] = x_ref[off:off + ROWS_PER_STREAM, :]

        barrier_sem = pltpu.get_barrier_semaphore()
        for nbr in (left, right):
            pl.semaphore_signal(
                barrier_sem, inc=1,
                device_id=(nbr,), device_id_type=pl.DeviceIdType.MESH,
            )
        pl.semaphore_wait(barrier_sem, 2)

        pending_sends = []

        def start_send(st, hop, origin):
            cw = st < STREAMS_PER_DIR
            desc = pltpu.make_async_remote_copy(
                src_ref=comm[st].at[origin],
                dst_ref=comm[st].at[origin],
                send_sem=send_sems[st].at[hop],
                recv_sem=recv_sems[st].at[hop],
                device_id=(right if cw else left,),
                device_id_type=pl.DeviceIdType.MESH,
            )
            desc.start()
            pending_sends.append(desc)

        for st in range(N_STREAMS):
            start_send(st, 0, my_pos)

        y = jnp.dot(x_ref[...], w_ref[...], preferred_element_type=jnp.float32)
        out_ref[pl.ds(my_pos * M_PER, M_PER), :] = _gelu(y)

        for h in range(HOPS):
            for st in range(N_STREAMS):
                cw = st < STREAMS_PER_DIR
                o_recv = _ring(my_slot - h - 1) if cw else _ring(my_slot + h + 1)
                recv = pltpu.make_async_remote_copy(
                    src_ref=comm[st].at[o_recv],
                    dst_ref=comm[st].at[o_recv],
                    send_sem=send_sems[st].at[h],
                    recv_sem=recv_sems[st].at[h],
                    device_id=(left if cw else right,),
                    device_id_type=pl.DeviceIdType.MESH,
                )
                recv.wait_recv()
                if h < HOPS - 1:
                    start_send(st, h + 1, o_recv)
                y = jnp.dot(comm[st][o_recv], w_ref[...],
                            preferred_element_type=jnp.float32)
                row = o_recv * M_PER + st * ROWS_PER_STREAM
                out_ref[pl.ds(row, ROWS_PER_STREAM), :] = _gelu(y)

        for desc in pending_sends:
            desc.wait_send()

    scratch = (
        [pltpu.VMEM((N_DEV, ROWS_PER_STREAM, K), jnp.float32)] * N_STREAMS
        + [pltpu.SemaphoreType.DMA((HOPS,))] * N_STREAMS
        + [pltpu.SemaphoreType.DMA((HOPS,))] * N_STREAMS
    )
    return pl.pallas_call(
        body,
        out_shape=jax.ShapeDtypeStruct((N_DEV * M_PER, N_PER), jnp.float32),
        in_specs=[
            pl.BlockSpec(memory_space=pltpu.VMEM),
            pl.BlockSpec(memory_space=pltpu.VMEM),
        ],
        out_specs=pl.BlockSpec(memory_space=pltpu.VMEM),
        scratch_shapes=scratch,
        compiler_params=pltpu.CompilerParams(collective_id=0),
    )(x, w_mat)


# device time: 28754 ns/iter; 2.0599x vs baseline; 1.0963x over previous
import jax
import jax.numpy as jnp
from jax import lax
from jax.experimental import pallas as pl
from jax.experimental.pallas import tpu as pltpu

N_DEV = 8
M_PER = 128
K = 1024
N_PER = 128

N_SUB = 4
SUB_ROWS = M_PER // N_SUB
HOPS = 4

_GELU_C = 0.7978845608028654


def _gelu(y):
    return 0.5 * y * (1.0 + jnp.tanh(_GELU_C * (y + 0.044715 * y * y * y)))


def _ring(s):
    s = s % N_DEV
    return jnp.where(s < 4, s, 11 - s)


_LAST_HOP_SUBS = {"cw": (0, 1), "ccw": (2, 3)}


def _subs_for_hop(direction, hop):
    return _LAST_HOP_SUBS[direction] if hop == HOPS - 1 else (0, 1, 2, 3)


def kernel(x, w_mat):
    def body(x_ref, w_ref, out_ref, comm_ref,
             cw_send, cw_recv, ccw_send, ccw_recv):
        my_pos = lax.axis_index("i")
        my_slot = _ring(my_pos)
        right = _ring(my_slot + 1)
        left = _ring(my_slot - 1)

        sems = {"cw": (cw_send, cw_recv), "ccw": (ccw_send, ccw_recv)}
        target = {"cw": right, "ccw": left}
        source = {"cw": left, "ccw": right}

        for j in range(N_SUB):
            comm_ref[my_pos, j, :, :] = x_ref[j * SUB_ROWS:(j + 1) * SUB_ROWS, :]

        barrier_sem = pltpu.get_barrier_semaphore()
        for nbr in (left, right):
            pl.semaphore_signal(
                barrier_sem, inc=1,
                device_id=(nbr,), device_id_type=pl.DeviceIdType.MESH,
            )
        pl.semaphore_wait(barrier_sem, 2)

        pending_sends = []

        def start_send(direction, hop, origin, sub):
            send_sem, recv_sem = sems[direction]
            desc = pltpu.make_async_remote_copy(
                src_ref=comm_ref.at[origin, sub],
                dst_ref=comm_ref.at[origin, sub],
                send_sem=send_sem.at[hop, sub],
                recv_sem=recv_sem.at[hop, sub],
                device_id=(target[direction],),
                device_id_type=pl.DeviceIdType.MESH,
            )
            desc.start()
            pending_sends.append(desc)

        for j in range(N_SUB):
            start_send("cw", 0, my_pos, j)
        for j in range(N_SUB):
            start_send("ccw", 0, my_pos, j)

        y = jnp.dot(x_ref[...], w_ref[...], preferred_element_type=jnp.float32)
        out_ref[pl.ds(my_pos * M_PER, M_PER), :] = _gelu(y)

        for h in range(HOPS):
            origins = {
                "cw": _ring(my_slot - h - 1),
                "ccw": _ring(my_slot + h + 1),
            }
            for j in range(N_SUB):
                for d in ("cw", "ccw"):
                    subs = _subs_for_hop(d, h)
                    if j >= len(subs):
                        continue
                    sub = subs[j]
                    send_sem, recv_sem = sems[d]
                    recv = pltpu.make_async_remote_copy(
                        src_ref=comm_ref.at[origins[d], sub],
                        dst_ref=comm_ref.at[origins[d], sub],
                        send_sem=send_sem.at[h, sub],
                        recv_sem=recv_sem.at[h, sub],
                        device_id=(source[d],),
                        device_id_type=pl.DeviceIdType.MESH,
                    )
                    recv.wait_recv()
                    if h + 1 < HOPS and sub in _subs_for_hop(d, h + 1):
                        start_send(d, h + 1, origins[d], sub)

            for d in ("cw", "ccw"):
                subs = _subs_for_hop(d, h)
                o = origins[d]
                if len(subs) == N_SUB:
                    xg = comm_ref[o]
                    y = jnp.dot(xg.reshape(M_PER, K), w_ref[...],
                                preferred_element_type=jnp.float32)
                    out_ref[pl.ds(o * M_PER, M_PER), :] = _gelu(y)
                else:
                    row0 = subs[0] * SUB_ROWS
                    xg = comm_ref[o, pl.ds(subs[0], len(subs))]
                    y = jnp.dot(xg.reshape(len(subs) * SUB_ROWS, K), w_ref[...],
                                preferred_element_type=jnp.float32)
                    out_ref[pl.ds(o * M_PER + row0, len(subs) * SUB_ROWS), :] = (
                        _gelu(y))

        for desc in pending_sends:
            desc.wait_send()

    return pl.pallas_call(
        body,
        out_shape=jax.ShapeDtypeStruct((N_DEV * M_PER, N_PER), jnp.float32),
        in_specs=[
            pl.BlockSpec(memory_space=pltpu.VMEM),
            pl.BlockSpec(memory_space=pltpu.VMEM),
        ],
        out_specs=pl.BlockSpec(memory_space=pltpu.VMEM),
        scratch_shapes=[
            pltpu.VMEM((N_DEV, N_SUB, SUB_ROWS, K), jnp.float32),
            pltpu.SemaphoreType.DMA((HOPS, N_SUB)),
            pltpu.SemaphoreType.DMA((HOPS, N_SUB)),
            pltpu.SemaphoreType.DMA((HOPS, N_SUB)),
            pltpu.SemaphoreType.DMA((HOPS, N_SUB)),
        ],
        compiler_params=pltpu.CompilerParams(collective_id=0),
    )(x, w_mat)
